# baseline (speedup 1.0000x reference)
import functools

import jax
import jax.numpy as jnp
from jax.experimental import pallas as pl
from jax.experimental.pallas import tpu as pltpu

BL = 512


def _bag_kernel(n_ref, x_ref, w1_ref, b1_ref, w2_ref, b2_ref, out_ref, acc_ref,
                *, num_j):
    b = pl.program_id(0)
    j = pl.program_id(1)
    n = n_ref[b]

    @pl.when(j == 0)
    def init():
        acc_ref[0:1, :] = jnp.zeros_like(acc_ref[0:1, :])

    @pl.when(j * BL < n)
    def compute():
        y = jnp.dot(x_ref[0].astype(jnp.bfloat16), w1_ref[...],
                    preferred_element_type=jnp.float32)
        y = jnp.maximum(y + b1_ref[...], 0.0)

        @pl.when((j + 1) * BL <= n)
        def full_block():
            acc_ref[0:1, :] += jnp.sum(y, axis=0, keepdims=True)

        @pl.when((j + 1) * BL > n)
        def boundary_block():
            row = j * BL + jax.lax.broadcasted_iota(jnp.int32, (BL, 1), 0)
            ym = jnp.where(row < n, y, 0.0)
            acc_ref[0:1, :] += jnp.sum(ym, axis=0, keepdims=True)

    @pl.when(j == num_j - 1)
    def finalize():
        pooled = acc_ref[0:1, :] / n.astype(jnp.float32)
        out = jnp.dot(pooled, w2_ref[...], preferred_element_type=jnp.float32)
        out_ref[0] = out + b2_ref[...]


def kernel(x, n_instances, W1, b1, W2, b2):
    B, L, D = x.shape
    DO = W2.shape[1]
    num_j = L // BL

    grid_spec = pltpu.PrefetchScalarGridSpec(
        num_scalar_prefetch=1,
        grid=(B, num_j),
        in_specs=[
            pl.BlockSpec(
                (1, BL, D),
                lambda b, j, n_ref: (b, jnp.minimum(j, pl.cdiv(n_ref[b], BL) - 1), 0),
            ),
            pl.BlockSpec((D, D), lambda b, j, n_ref: (0, 0)),
            pl.BlockSpec((1, D), lambda b, j, n_ref: (0, 0)),
            pl.BlockSpec((D, DO), lambda b, j, n_ref: (0, 0)),
            pl.BlockSpec((1, DO), lambda b, j, n_ref: (0, 0)),
        ],
        out_specs=pl.BlockSpec((1, 1, DO), lambda b, j, n_ref: (b, 0, 0)),
        scratch_shapes=[pltpu.VMEM((8, D), jnp.float32)],
    )

    out = pl.pallas_call(
        functools.partial(_bag_kernel, num_j=num_j),
        grid_spec=grid_spec,
        out_shape=jax.ShapeDtypeStruct((B, 1, DO), jnp.float32),
        compiler_params=pltpu.CompilerParams(
            dimension_semantics=("arbitrary", "arbitrary"),
        ),
    )(n_instances, x, W1.astype(jnp.bfloat16), b1.reshape(1, D), W2,
      b2.reshape(1, DO))
    return out.reshape(B, DO)


# speedup vs baseline: 1.0720x; 1.0720x over previous
import functools

import jax
import jax.numpy as jnp
from jax.experimental import pallas as pl
from jax.experimental.pallas import tpu as pltpu

BL = 512


def _bag_kernel(n_ref, x_ref, w1_ref, b1_ref, w2_ref, b2_ref, out_ref, acc_ref,
                *, num_j):
    b = pl.program_id(0)
    j = pl.program_id(1)
    n = n_ref[b]

    @pl.when(j * BL < n)
    def compute():
        y = jnp.dot(x_ref[0].astype(jnp.bfloat16), w1_ref[...],
                    preferred_element_type=jnp.float32)
        y = jnp.maximum(y + b1_ref[...], 0.0)
        row = j * BL + jax.lax.broadcasted_iota(jnp.int32, (BL, 1), 0)
        y = jnp.where(row < n, y, 0.0)
        s = jnp.sum(y, axis=0, keepdims=True)

        @pl.when(j == 0)
        def init():
            acc_ref[0:1, :] = s

        @pl.when(j != 0)
        def add():
            acc_ref[0:1, :] = acc_ref[0:1, :] + s

    @pl.when(j == num_j - 1)
    def finalize():
        pooled = acc_ref[0:1, :] / n.astype(jnp.float32)
        out = jnp.dot(pooled, w2_ref[...], preferred_element_type=jnp.float32)
        out_ref[0] = out + b2_ref[...]


def kernel(x, n_instances, W1, b1, W2, b2):
    B, L, D = x.shape
    DO = W2.shape[1]
    num_j = L // BL

    grid_spec = pltpu.PrefetchScalarGridSpec(
        num_scalar_prefetch=1,
        grid=(B, num_j),
        in_specs=[
            pl.BlockSpec(
                (1, BL, D),
                lambda b, j, n_ref: (b, jnp.minimum(j, pl.cdiv(n_ref[b], BL) - 1), 0),
            ),
            pl.BlockSpec((D, D), lambda b, j, n_ref: (0, 0)),
            pl.BlockSpec((1, D), lambda b, j, n_ref: (0, 0)),
            pl.BlockSpec((D, DO), lambda b, j, n_ref: (0, 0)),
            pl.BlockSpec((1, DO), lambda b, j, n_ref: (0, 0)),
        ],
        out_specs=pl.BlockSpec((1, 1, DO), lambda b, j, n_ref: (b, 0, 0)),
        scratch_shapes=[pltpu.VMEM((8, D), jnp.float32)],
    )

    out = pl.pallas_call(
        functools.partial(_bag_kernel, num_j=num_j),
        grid_spec=grid_spec,
        out_shape=jax.ShapeDtypeStruct((B, 1, DO), jnp.float32),
        compiler_params=pltpu.CompilerParams(
            dimension_semantics=("arbitrary", "arbitrary"),
        ),
    )(n_instances, x, W1.astype(jnp.bfloat16), b1.reshape(1, D), W2,
      b2.reshape(1, DO))
    return out.reshape(B, DO)
